# baseline (speedup 1.0000x reference)
import jax
import jax.numpy as jnp
from jax import lax
from jax.experimental import pallas as pl
from jax.experimental.pallas import tpu as pltpu
from jax.experimental.pallas import tpu_sc as plsc

EMBED_DIM = 232
NUM_SYMBOLS = 100
ROWS_PER_WORKER = 8
NUM_FULL_WORKERS = NUM_SYMBOLS // ROWS_PER_WORKER
TAIL_ROWS = NUM_SYMBOLS - NUM_FULL_WORKERS * ROWS_PER_WORKER


def _gather_body(x_hbm, table_hbm, out_hbm, idx_v, rows_v, sem):
    wid = lax.axis_index("s") * 2 + lax.axis_index("c")

    @pl.when(wid <= NUM_FULL_WORKERS)
    def _():
        base = wid * ROWS_PER_WORKER
        pltpu.sync_copy(x_hbm.at[0, pl.ds(base, ROWS_PER_WORKER)], idx_v)
        pltpu.async_copy(table_hbm.at[idx_v], rows_v, sem).wait()

    @pl.when(wid < NUM_FULL_WORKERS)
    def _():
        base = wid * ROWS_PER_WORKER
        pltpu.sync_copy(rows_v, out_hbm.at[pl.ds(base, ROWS_PER_WORKER)])

    @pl.when(wid == NUM_FULL_WORKERS)
    def _():
        base = NUM_FULL_WORKERS * ROWS_PER_WORKER
        pltpu.sync_copy(rows_v.at[pl.ds(0, TAIL_ROWS)],
                        out_hbm.at[pl.ds(base, TAIL_ROWS)])


def kernel(x, table):
    mesh = plsc.VectorSubcoreMesh(core_axis_name="c", subcore_axis_name="s")
    return pl.kernel(
        _gather_body,
        out_type=jax.ShapeDtypeStruct((NUM_SYMBOLS, EMBED_DIM), jnp.float32),
        mesh=mesh,
        scratch_types=[
            pltpu.VMEM((ROWS_PER_WORKER,), jnp.int32),
            pltpu.VMEM((ROWS_PER_WORKER, EMBED_DIM), jnp.float32),
            pltpu.SemaphoreType.DMA,
        ],
        compiler_params=pltpu.CompilerParams(use_tc_tiling_on_sc=False),
    )(x, table)


# speedup vs baseline: 2.3411x; 2.3411x over previous
import jax
import jax.numpy as jnp
from jax import lax
from jax.experimental import pallas as pl
from jax.experimental.pallas import tpu as pltpu
from jax.experimental.pallas import tpu_sc as plsc

EMBED_DIM = 232
NUM_SYMBOLS = 100
ROWS_PER_WORKER = 8
NUM_WORKERS = 13
PAD_ROWS = NUM_WORKERS * ROWS_PER_WORKER
CHUNK_STARTS = tuple(range(0, 224, 16)) + (EMBED_DIM - 16,)


def _gather_body(x_hbm, table_hbm, out_hbm, xblk_v, grp_v, rows_v, sem):
    wid = lax.axis_index("s") * 2 + lax.axis_index("c")

    @pl.when(wid < NUM_WORKERS)
    def _():
        base = wid * ROWS_PER_WORKER
        pltpu.sync_copy(x_hbm.at[pl.ds(0, 8), pl.ds(0, 128)], xblk_v)
        idx = xblk_v[0, pl.ds(base, 16)]
        for j in range(ROWS_PER_WORKER):
            r = idx[j]
            g8 = pl.multiple_of((r >> 3) << 3, 8)
            pltpu.async_copy(table_hbm.at[pl.ds(g8, 8)], grp_v.at[j], sem)
        for j in range(ROWS_PER_WORKER):
            pltpu.make_async_copy(
                table_hbm.at[pl.ds(0, 8)], grp_v.at[j], sem).wait()
        for j in range(ROWS_PER_WORKER):
            r = idx[j]
            sub = r & 7
            for start in CHUNK_STARTS:
                rows_v[j, pl.ds(start, 16)] = grp_v[j, sub, pl.ds(start, 16)]
        pltpu.sync_copy(rows_v, out_hbm.at[pl.ds(base, ROWS_PER_WORKER)])


def kernel(x, table):
    mesh = plsc.VectorSubcoreMesh(core_axis_name="c", subcore_axis_name="s")
    out = pl.kernel(
        _gather_body,
        out_type=jax.ShapeDtypeStruct((PAD_ROWS, EMBED_DIM), jnp.float32),
        mesh=mesh,
        scratch_types=[
            pltpu.VMEM((8, 128), jnp.int32),
            pltpu.VMEM((ROWS_PER_WORKER, 8, EMBED_DIM), jnp.float32),
            pltpu.VMEM((ROWS_PER_WORKER, EMBED_DIM), jnp.float32),
            pltpu.SemaphoreType.DMA,
        ],
    )(x, table)
    return out[:NUM_SYMBOLS]


# speedup vs baseline: 2.4447x; 1.0443x over previous
import jax
import jax.numpy as jnp
from jax import lax
from jax.experimental import pallas as pl
from jax.experimental.pallas import tpu as pltpu
from jax.experimental.pallas import tpu_sc as plsc

EMBED_DIM = 232
NUM_SYMBOLS = 100
ROWS_PER_WORKER = 8
NUM_WORKERS = 13
PAD_ROWS = NUM_WORKERS * ROWS_PER_WORKER
CHUNK_STARTS = tuple(range(0, 224, 16)) + (EMBED_DIM - 16,)


def _gather_body(x_hbm, table_hbm, out_hbm, xblk_v, grp_v, rows_v, sem):
    wid = lax.axis_index("s")

    @pl.when(wid < NUM_WORKERS)
    def _():
        base = wid * ROWS_PER_WORKER
        pltpu.sync_copy(x_hbm.at[pl.ds(0, 8), pl.ds(0, 128)], xblk_v)
        idx = xblk_v[0, pl.ds(base, 16)]
        for j in range(ROWS_PER_WORKER):
            r = idx[j]
            g8 = pl.multiple_of((r >> 3) << 3, 8)
            pltpu.async_copy(table_hbm.at[pl.ds(g8, 8)], grp_v.at[j], sem)
        for j in range(ROWS_PER_WORKER):
            pltpu.make_async_copy(
                table_hbm.at[pl.ds(0, 8)], grp_v.at[j], sem).wait()
        for j in range(ROWS_PER_WORKER):
            r = idx[j]
            sub = r & 7
            for start in CHUNK_STARTS:
                rows_v[j, pl.ds(start, 16)] = grp_v[j, sub, pl.ds(start, 16)]
        pltpu.sync_copy(rows_v, out_hbm.at[pl.ds(base, ROWS_PER_WORKER)])


def kernel(x, table):
    mesh = plsc.VectorSubcoreMesh(core_axis_name="c", subcore_axis_name="s", num_cores=1)
    out = pl.kernel(
        _gather_body,
        out_type=jax.ShapeDtypeStruct((PAD_ROWS, EMBED_DIM), jnp.float32),
        mesh=mesh,
        scratch_types=[
            pltpu.VMEM((8, 128), jnp.int32),
            pltpu.VMEM((ROWS_PER_WORKER, 8, EMBED_DIM), jnp.float32),
            pltpu.VMEM((ROWS_PER_WORKER, EMBED_DIM), jnp.float32),
            pltpu.SemaphoreType.DMA,
        ],
    )(x, table)
    return out[:NUM_SYMBOLS]


# speedup vs baseline: 2.4580x; 1.0055x over previous
import jax
import jax.numpy as jnp
from jax import lax
from jax.experimental import pallas as pl
from jax.experimental.pallas import tpu as pltpu
from jax.experimental.pallas import tpu_sc as plsc

EMBED_DIM = 232
NUM_SYMBOLS = 100
ROWS_PER_WORKER = 8
NUM_WORKERS = 13
PAD_ROWS = NUM_WORKERS * ROWS_PER_WORKER
CHUNK_STARTS = tuple(range(0, 224, 16)) + (EMBED_DIM - 16,)


def _gather_body(x_hbm, table_hbm, out_hbm, xblk_v, grp_v, rows_v, sem):
    wid = lax.axis_index("s")

    @pl.when(wid < NUM_WORKERS)
    def _():
        base = wid * ROWS_PER_WORKER
        pltpu.sync_copy(x_hbm.at[pl.ds(0, 8), pl.ds(0, 128)], xblk_v)
        idx = xblk_v[0, pl.ds(base, 16)]
        for j in range(ROWS_PER_WORKER):
            r = idx[j]
            g8 = pl.multiple_of((r >> 3) << 3, 8)
            pltpu.async_copy(table_hbm.at[pl.ds(g8, 8)], grp_v.at[j], sem)
        for j in range(ROWS_PER_WORKER):
            pltpu.make_async_copy(
                table_hbm.at[pl.ds(0, 8)], grp_v.at[j], sem).wait()
            r = idx[j]
            sub = r & 7
            for start in CHUNK_STARTS:
                rows_v[j, pl.ds(start, 16)] = grp_v[j, sub, pl.ds(start, 16)]
        pltpu.sync_copy(rows_v, out_hbm.at[pl.ds(base, ROWS_PER_WORKER)])


def kernel(x, table):
    mesh = plsc.VectorSubcoreMesh(core_axis_name="c", subcore_axis_name="s", num_cores=1)
    out = pl.kernel(
        _gather_body,
        out_type=jax.ShapeDtypeStruct((PAD_ROWS, EMBED_DIM), jnp.float32),
        mesh=mesh,
        scratch_types=[
            pltpu.VMEM((8, 128), jnp.int32),
            pltpu.VMEM((ROWS_PER_WORKER, 8, EMBED_DIM), jnp.float32),
            pltpu.VMEM((ROWS_PER_WORKER, EMBED_DIM), jnp.float32),
            pltpu.SemaphoreType.DMA,
        ],
    )(x, table)
    return out[:NUM_SYMBOLS]
